# Initial kernel scaffold; baseline (speedup 1.0000x reference)
import functools

import jax
import jax.numpy as jnp
from jax import lax
from jax.experimental import pallas as pl
from jax.experimental.pallas import tpu as pltpu
from jax.experimental.pallas import tpu_sc as plsc

VOCAB_SIZE = 100000
EMBED_DIM = 128
BATCH = 1024
SEQ = 200

_INFO = plsc.get_sparse_core_info()
_NC = _INFO.num_cores
_NS = _INFO.num_subcores
_NW = _NC * _NS
_N_TOTAL = BATCH * SEQ
_N_PER_W = _N_TOTAL // _NW
_CHUNK = 128
_N_CHUNKS = _N_PER_W // _CHUNK


def _gather_body(idx_hbm, table_hbm, out_hbm, idx_v, rows_v, sem):
    wid = lax.axis_index("s") * _NC + lax.axis_index("c")
    base = wid * _N_PER_W
    pltpu.sync_copy(idx_hbm.at[pl.ds(base, _N_PER_W)], idx_v)

    def step(j, carry):
        off = j * _CHUNK
        pltpu.async_copy(
            table_hbm.at[idx_v.at[pl.ds(off, _CHUNK)]], rows_v, sem
        ).wait()
        pltpu.sync_copy(rows_v, out_hbm.at[pl.ds(base + off, _CHUNK)])
        return carry

    lax.fori_loop(0, _N_CHUNKS, step, 0)


_gather = functools.partial(
    pl.kernel,
    mesh=plsc.VectorSubcoreMesh(core_axis_name="c", subcore_axis_name="s"),
    out_type=jax.ShapeDtypeStruct((_N_TOTAL, EMBED_DIM), jnp.float32),
    scratch_types=[
        pltpu.VMEM((_N_PER_W,), jnp.int32),
        pltpu.VMEM((_CHUNK, EMBED_DIM), jnp.float32),
        pltpu.SemaphoreType.DMA,
    ],
)(_gather_body)


def kernel(input_ids, embedding):
    ids_flat = jnp.reshape(input_ids.astype(jnp.int32), (_N_TOTAL,))
    out = _gather(ids_flat, embedding)
    return jnp.reshape(out, (BATCH, SEQ, EMBED_DIM))


# speedup vs baseline: 5.7523x; 5.7523x over previous
import functools

import jax
import jax.numpy as jnp
from jax import lax
from jax.experimental import pallas as pl
from jax.experimental.pallas import tpu as pltpu
from jax.experimental.pallas import tpu_sc as plsc

VOCAB_SIZE = 100000
EMBED_DIM = 128
BATCH = 1024
SEQ = 200

_NC = 2
_NS = 16
_NW = _NC * _NS
_N_TOTAL = BATCH * SEQ
_N_PER_W = _N_TOTAL // _NW
_CHUNK = 128
_N_CHUNKS = _N_PER_W // _CHUNK


def _gather_body(idx_hbm, table_hbm, out_hbm, idx_v, rows_v, sem):
    wid = lax.axis_index("s") * _NC + lax.axis_index("c")
    base = wid * _N_PER_W
    pltpu.sync_copy(idx_hbm.at[pl.ds(base, _N_PER_W)], idx_v)

    def step(j, carry):
        off = j * _CHUNK
        pltpu.async_copy(
            table_hbm.at[idx_v.at[pl.ds(off, _CHUNK)]], rows_v, sem
        ).wait()
        pltpu.sync_copy(rows_v, out_hbm.at[pl.ds(base + off, _CHUNK)])
        return carry

    lax.fori_loop(0, _N_CHUNKS, step, 0)


@functools.lru_cache(maxsize=1)
def _build_gather():
    return functools.partial(
        pl.kernel,
        mesh=plsc.VectorSubcoreMesh(core_axis_name="c", subcore_axis_name="s"),
        out_type=jax.ShapeDtypeStruct((_N_TOTAL, EMBED_DIM), jnp.float32),
        scratch_types=[
            pltpu.VMEM((_N_PER_W,), jnp.int32),
            pltpu.VMEM((_CHUNK, EMBED_DIM), jnp.float32),
            pltpu.SemaphoreType.DMA,
        ],
    )(_gather_body)


def kernel(input_ids, embedding):
    ids_flat = jnp.reshape(input_ids.astype(jnp.int32), (_N_TOTAL,))
    out = _build_gather()(ids_flat, embedding)
    return jnp.reshape(out, (BATCH, SEQ, EMBED_DIM))


# speedup vs baseline: 7.9230x; 1.3773x over previous
import functools

import jax
import jax.numpy as jnp
from jax import lax
from jax.experimental import pallas as pl
from jax.experimental.pallas import tpu as pltpu
from jax.experimental.pallas import tpu_sc as plsc

VOCAB_SIZE = 100000
EMBED_DIM = 128
BATCH = 1024
SEQ = 200

_NC = 2
_NS = 16
_NW = _NC * _NS
_N_TOTAL = BATCH * SEQ
_N_PER_W = _N_TOTAL // _NW
_CHUNK = 128
_N_CHUNKS = _N_PER_W // _CHUNK


def _gather_body(idx_hbm, table_hbm, out_hbm, idx_v, rows_v0, rows_v1, sem0, sem1):
    wid = lax.axis_index("s") * _NC + lax.axis_index("c")
    base = wid * _N_PER_W
    pltpu.sync_copy(idx_hbm.at[pl.ds(base, _N_PER_W)], idx_v)

    rows = (rows_v0, rows_v1)
    sems = (sem0, sem1)

    def gather_src(j):
        return table_hbm.at[idx_v.at[pl.ds(j * _CHUNK, _CHUNK)]]

    pltpu.make_async_copy(gather_src(0), rows[0], sems[0]).start()

    def step2(g, carry):
        for b in range(2):
            j = 2 * g + b
            nb = (b + 1) % 2

            @pl.when(j + 1 < _N_CHUNKS)
            def _():
                pltpu.make_async_copy(gather_src(j + 1), rows[nb], sems[nb]).start()

            pltpu.make_async_copy(gather_src(j), rows[b], sems[b]).wait()
            pltpu.sync_copy(rows[b], out_hbm.at[pl.ds(base + j * _CHUNK, _CHUNK)])
        return carry

    lax.fori_loop(0, _N_CHUNKS // 2, step2, 0)


@functools.lru_cache(maxsize=1)
def _build_gather():
    return functools.partial(
        pl.kernel,
        mesh=plsc.VectorSubcoreMesh(core_axis_name="c", subcore_axis_name="s"),
        out_type=jax.ShapeDtypeStruct((_N_TOTAL, EMBED_DIM), jnp.float32),
        scratch_types=[
            pltpu.VMEM((_N_PER_W,), jnp.int32),
            pltpu.VMEM((_CHUNK, EMBED_DIM), jnp.float32),
            pltpu.VMEM((_CHUNK, EMBED_DIM), jnp.float32),
            pltpu.SemaphoreType.DMA,
            pltpu.SemaphoreType.DMA,
        ],
    )(_gather_body)


def kernel(input_ids, embedding):
    ids_flat = jnp.reshape(input_ids.astype(jnp.int32), (_N_TOTAL,))
    out = _build_gather()(ids_flat, embedding)
    return jnp.reshape(out, (BATCH, SEQ, EMBED_DIM))


# speedup vs baseline: 8.0244x; 1.0128x over previous
import functools

import jax
import jax.numpy as jnp
from jax import lax
from jax.experimental import pallas as pl
from jax.experimental.pallas import tpu as pltpu
from jax.experimental.pallas import tpu_sc as plsc

VOCAB_SIZE = 100000
EMBED_DIM = 128
BATCH = 1024
SEQ = 200

_NC = 2
_NS = 16
_NW = _NC * _NS
_N_TOTAL = BATCH * SEQ
_N_PER_W = _N_TOTAL // _NW
_CHUNK = 128
_N_CHUNKS = _N_PER_W // _CHUNK


_NBUF = 5


def _gather_body(idx_hbm, table_hbm, out_hbm, idx_v, *scratch):
    rows = scratch[:_NBUF]
    gsems = scratch[_NBUF:2 * _NBUF]
    wsems = scratch[2 * _NBUF:3 * _NBUF]

    wid = lax.axis_index("s") * _NC + lax.axis_index("c")
    base = wid * _N_PER_W
    pltpu.sync_copy(idx_hbm.at[pl.ds(base, _N_PER_W)], idx_v)

    def gather(j, b):
        return pltpu.make_async_copy(
            table_hbm.at[idx_v.at[pl.ds(j * _CHUNK, _CHUNK)]], rows[b], gsems[b]
        )

    def write(j, b):
        return pltpu.make_async_copy(
            rows[b], out_hbm.at[pl.ds(base + j * _CHUNK, _CHUNK)], wsems[b]
        )

    gather(0, 0).start()
    gather(1, 1).start()

    def group(g, carry):
        for b in range(_NBUF):
            j = g * _NBUF + b
            nb = (b + 2) % _NBUF

            @pl.when(j >= 3)
            def _():
                write(j - 3, nb).wait()

            @pl.when(j + 2 < _N_CHUNKS)
            def _():
                gather(j + 2, nb).start()

            gather(j, b).wait()
            write(j, b).start()
        return carry

    lax.fori_loop(0, _N_CHUNKS // _NBUF, group, 0)

    for j in range(_N_CHUNKS - 3, _N_CHUNKS):
        write(j, j % _NBUF).wait()


@functools.lru_cache(maxsize=1)
def _build_gather():
    return functools.partial(
        pl.kernel,
        mesh=plsc.VectorSubcoreMesh(core_axis_name="c", subcore_axis_name="s"),
        out_type=jax.ShapeDtypeStruct((_N_TOTAL, EMBED_DIM), jnp.float32),
        scratch_types=(
            [pltpu.VMEM((_N_PER_W,), jnp.int32)]
            + [pltpu.VMEM((_CHUNK, EMBED_DIM), jnp.float32)] * _NBUF
            + [pltpu.SemaphoreType.DMA] * (2 * _NBUF)
        ),
    )(_gather_body)


def kernel(input_ids, embedding):
    ids_flat = jnp.reshape(input_ids.astype(jnp.int32), (_N_TOTAL,))
    out = _build_gather()(ids_flat, embedding)
    return jnp.reshape(out, (BATCH, SEQ, EMBED_DIM))


# speedup vs baseline: 8.0248x; 1.0001x over previous
import functools

import jax
import jax.numpy as jnp
from jax import lax
from jax.experimental import pallas as pl
from jax.experimental.pallas import tpu as pltpu
from jax.experimental.pallas import tpu_sc as plsc

VOCAB_SIZE = 100000
EMBED_DIM = 128
BATCH = 1024
SEQ = 200

_NC = 2
_NS = 16
_NW = _NC * _NS
_N_TOTAL = BATCH * SEQ
_N_PER_W = _N_TOTAL // _NW
_CHUNK = 160
_N_CHUNKS = _N_PER_W // _CHUNK


_NBUF = 5


def _gather_body(idx_hbm, table_hbm, out_hbm, idx_v, *scratch):
    rows = scratch[:_NBUF]
    gsems = scratch[_NBUF:2 * _NBUF]
    wsems = scratch[2 * _NBUF:3 * _NBUF]

    wid = lax.axis_index("s") * _NC + lax.axis_index("c")
    base = wid * _N_PER_W
    pltpu.sync_copy(idx_hbm.at[pl.ds(base, _N_PER_W)], idx_v)

    def gather(j, b):
        return pltpu.make_async_copy(
            table_hbm.at[idx_v.at[pl.ds(j * _CHUNK, _CHUNK)]], rows[b], gsems[b]
        )

    def write(j, b):
        return pltpu.make_async_copy(
            rows[b], out_hbm.at[pl.ds(base + j * _CHUNK, _CHUNK)], wsems[b]
        )

    gather(0, 0).start()
    gather(1, 1).start()

    def group(g, carry):
        for b in range(_NBUF):
            j = g * _NBUF + b
            nb = (b + 2) % _NBUF

            @pl.when(j >= 3)
            def _():
                write(j - 3, nb).wait()

            @pl.when(j + 2 < _N_CHUNKS)
            def _():
                gather(j + 2, nb).start()

            gather(j, b).wait()
            write(j, b).start()
        return carry

    lax.fori_loop(0, _N_CHUNKS // _NBUF, group, 0)

    for j in range(_N_CHUNKS - 3, _N_CHUNKS):
        write(j, j % _NBUF).wait()


@functools.lru_cache(maxsize=1)
def _build_gather():
    return functools.partial(
        pl.kernel,
        mesh=plsc.VectorSubcoreMesh(core_axis_name="c", subcore_axis_name="s"),
        out_type=jax.ShapeDtypeStruct((_N_TOTAL, EMBED_DIM), jnp.float32),
        scratch_types=(
            [pltpu.VMEM((_N_PER_W,), jnp.int32)]
            + [pltpu.VMEM((_CHUNK, EMBED_DIM), jnp.float32)] * _NBUF
            + [pltpu.SemaphoreType.DMA] * (2 * _NBUF)
        ),
    )(_gather_body)


def kernel(input_ids, embedding):
    ids_flat = jnp.reshape(input_ids.astype(jnp.int32), (_N_TOTAL,))
    out = _build_gather()(ids_flat, embedding)
    return jnp.reshape(out, (BATCH, SEQ, EMBED_DIM))
